# baseline (speedup 1.0000x reference)
import functools

import jax
import jax.numpy as jnp
from jax import lax
from jax.experimental import pallas as pl
from jax.experimental.pallas import tpu as pltpu
from jax.experimental.pallas import tpu_sc as plsc

B = 16384
D = 32
NC, NS, L = 2, 16, 16
NW = NC * NS
B_PER_W = B // NW
CHUNK = 128
NCHUNK = B_PER_W // CHUNK
GROUPS = B_PER_W // L


def _body(vid_hbm, gid_hbm, vtab_hbm, gtab_hbm, wv_hbm, bv_hbm, out_hbm,
          vidx, gidx, vrows, grows, wv, bv, outs, sem):
    wid = lax.axis_index("s") * NC + lax.axis_index("c")
    cbase = wid * NCHUNK

    pltpu.sync_copy(vid_hbm.at[pl.ds(cbase, NCHUNK)], vidx)
    pltpu.sync_copy(gid_hbm.at[pl.ds(cbase, NCHUNK)], gidx)
    pltpu.sync_copy(wv_hbm, wv)
    pltpu.sync_copy(bv_hbm, bv)

    copies = []
    for j in range(NCHUNK):
        copies.append(pltpu.async_copy(
            vtab_hbm.at[vidx.at[j]], vrows.at[pl.ds(j * CHUNK, CHUNK)], sem))
        copies.append(pltpu.async_copy(
            gtab_hbm.at[gidx.at[j]], grows.at[pl.ds(j * CHUNK, CHUNK)], sem))
    for c in copies:
        c.wait()

    lanes = lax.iota(jnp.int32, L)
    w = wv[...]
    bb = bv[...]

    def group_body(g, carry):
        row_idx = g * L + lanes
        dot = jnp.zeros((L,), jnp.float32)
        mm = jnp.zeros((L,), jnp.float32)
        gg = jnp.zeros((L,), jnp.float32)
        for k in range(D):
            col = jnp.bitwise_and(lanes + k, D - 1)
            m = plsc.load_gather(vrows, [row_idx, col])
            ge = plsc.load_gather(grows, [row_idx, col])
            dot = dot + m * ge
            mm = mm + m * m
            gg = gg + ge * ge
        x = jnp.maximum(mm, 1e-12) * jnp.maximum(gg, 1e-12)
        i = plsc.bitcast(x, jnp.int32)
        y = plsc.bitcast(jnp.int32(0x5F3759DF) - (i >> 1), jnp.float32)
        for _ in range(3):
            y = y * (1.5 - 0.5 * x * y * y)
        logit = dot * y * w + bb
        prob = 1.0 / (1.0 + jnp.exp(-logit))
        outs[pl.ds(g * L, L)] = prob
        return carry

    lax.fori_loop(0, GROUPS, group_body, 0)
    pltpu.sync_copy(outs, out_hbm.at[pl.ds(wid * B_PER_W, B_PER_W)])


@jax.jit
def _run(vid, gid, vtab, gtab, wv, bv):
    mesh = plsc.VectorSubcoreMesh(
        core_axis_name="c", subcore_axis_name="s",
        num_cores=NC, num_subcores=NS)
    f = functools.partial(
        pl.kernel,
        out_type=jax.ShapeDtypeStruct((B,), jnp.float32),
        mesh=mesh,
        compiler_params=pltpu.CompilerParams(
            needs_layout_passes=False, use_tc_tiling_on_sc=False),
        scratch_types=[
            pltpu.VMEM((NCHUNK, CHUNK), jnp.int32),
            pltpu.VMEM((NCHUNK, CHUNK), jnp.int32),
            pltpu.VMEM((B_PER_W, D), jnp.float32),
            pltpu.VMEM((B_PER_W, D), jnp.float32),
            pltpu.VMEM((L,), jnp.float32),
            pltpu.VMEM((L,), jnp.float32),
            pltpu.VMEM((B_PER_W,), jnp.float32),
            pltpu.SemaphoreType.DMA,
        ],
    )(_body)
    return f(vid, gid, vtab, gtab, wv, bv)


def kernel(video_ids, genre_ids, video_table, genre_table, W, b):
    vid = video_ids.astype(jnp.int32).reshape(NW * NCHUNK, CHUNK)
    gid = genre_ids.astype(jnp.int32).reshape(NW * NCHUNK, CHUNK)
    wv = jnp.full((L,), W[0, 0], dtype=jnp.float32)
    bv = jnp.full((L,), b[0], dtype=jnp.float32)
    out = _run(vid, gid, video_table, genre_table, wv, bv)
    return out.reshape(B, 1)
